# baseline (speedup 1.0000x reference)
import jax
import jax.numpy as jnp
from jax import lax
from jax.experimental import pallas as pl
from jax.experimental.pallas import tpu as pltpu
from jax.experimental.pallas import tpu_sc as plsc

NN = 10000
EE = 160000
DH = 128
NCORES = 2
NSUB = 16
EDGES_PER_TILE = EE // NSUB
KB = 80
EPT_PAD = 10080
NBATCH = EPT_PAD // KB
ROWS_PER_TILE = 624
ROWS_LAST = NN - 15 * ROWS_PER_TILE
FIN_CHUNK = 16
NFIN = ROWS_PER_TILE // FIN_CHUNK


def _hop(table_ref, src4_ref, dst4_ref, acc, valbuf, sidx, didx, rows,
         gsem, ssem, xsem, dsem, c, s):
    mrow = (c * NSUB + s) * NBATCH

    def issue_meta(b, k):
        pltpu.async_copy(src4_ref.at[mrow + b], sidx[k], xsem[k])
        pltpu.async_copy(dst4_ref.at[s * NBATCH + b], didx[k], dsem[k])

    def wait_sidx(b, k):
        pltpu.make_async_copy(src4_ref.at[mrow + b], sidx[k],
                              xsem[k]).wait()

    def wait_didx(b, k):
        pltpu.make_async_copy(dst4_ref.at[s * NBATCH + b], didx[k],
                              dsem[k]).wait()

    def start_gather(b, k):
        wait_sidx(b, k)
        pltpu.async_copy(table_ref.at[sidx[k].at[0]], rows[k], gsem[k])

    def wait_gather(b, k):
        pltpu.make_async_copy(table_ref.at[sidx[k].at[0]], rows[k],
                              gsem[k]).wait()

    def scale(b, k):
        def grp(g, carry):
            base = pl.multiple_of(g * 16, 16)
            v16 = valbuf[pl.ds(b * KB + base, 16)]
            for e in range(16):
                r = base + e
                v = v16[e]
                for q in range(DH // 16):
                    sl = pl.ds(q * 16, 16)
                    rows[k][r, sl] = rows[k][r, sl] * v
            return carry

        lax.fori_loop(0, KB // 16, grp, 0)

    def start_scatter(b, k):
        wait_didx(b, k)
        pltpu.async_copy(rows[k], acc.at[didx[k].at[0]], ssem[k], add=True)

    def wait_scatter(b, k):
        pltpu.make_async_copy(rows[k], acc.at[didx[k].at[0]],
                              ssem[k]).wait()

    issue_meta(0, 0)
    issue_meta(1, 1)
    start_gather(0, 0)

    def sub(p, b, k):
        k1 = (k + 1) % 3
        k2 = (k + 2) % 3
        wait_gather(b, k)

        @pl.when(b + 1 < NBATCH)
        def _():
            start_gather(b + 1, k1)

        scale(b, k)
        start_scatter(b, k)

        @pl.when(b > 0)
        def _():
            wait_scatter(b - 1, k2)

        @pl.when(b + 2 < NBATCH)
        def _():
            issue_meta(b + 2, k2)

    def triple(p, carry):
        b0 = 3 * p
        sub(p, b0, 0)
        sub(p, b0 + 1, 1)
        sub(p, b0 + 2, 2)
        return carry

    lax.fori_loop(0, NBATCH // 3, triple, 0)

    wait_scatter(NBATCH - 1, (NBATCH - 1) % 3)


def _body(xs_ref, src4_ref, dst4_ref, vals2_ref, alpha_ref, zeros_ref,
          out_ref, ax_ref,
          acc, valbuf, sidx0, sidx1, sidx2, didx0, didx1, didx2,
          rows0, rows1, rows2, avec,
          gsem0, gsem1, gsem2, ssem0, ssem1, ssem2,
          xsem0, xsem1, xsem2, dsem0, dsem1, dsem2):
    rows = (rows0, rows1, rows2)
    sidx = (sidx0, sidx1, sidx2)
    didx = (didx0, didx1, didx2)
    gsem = (gsem0, gsem1, gsem2)
    ssem = (ssem0, ssem1, ssem2)
    xsem = (xsem0, xsem1, xsem2)
    dsem = (dsem0, dsem1, dsem2)
    c = lax.axis_index("c")
    s = lax.axis_index("s")
    coff = c * NN
    stripe = s * ROWS_PER_TILE
    is_last = s == NSUB - 1

    def _zero_acc():
        pltpu.sync_copy(zeros_ref.at[pl.ds(0, ROWS_PER_TILE)],
                        acc.at[pl.ds(stripe, ROWS_PER_TILE)])

        @pl.when(is_last)
        def _():
            pltpu.sync_copy(
                zeros_ref.at[pl.ds(0, ROWS_LAST - ROWS_PER_TILE)],
                acc.at[pl.ds(stripe + ROWS_PER_TILE,
                             ROWS_LAST - ROWS_PER_TILE)])

    with jax.named_scope("stage_in"):
        pltpu.sync_copy(alpha_ref, avec)
        pltpu.sync_copy(vals2_ref.at[s], valbuf)
        _zero_acc()
        plsc.subcore_barrier()

    with jax.named_scope("hop1"):
        _hop(xs_ref, src4_ref, dst4_ref, acc, valbuf, sidx, didx, rows,
             gsem, ssem, xsem, dsem, c, s)
        plsc.subcore_barrier()

    with jax.named_scope("ax_out"):
        pltpu.sync_copy(acc.at[pl.ds(stripe, ROWS_PER_TILE)],
                        ax_ref.at[pl.ds(coff + stripe, ROWS_PER_TILE)])

        @pl.when(is_last)
        def _():
            pltpu.sync_copy(
                acc.at[pl.ds(stripe + ROWS_PER_TILE,
                             ROWS_LAST - ROWS_PER_TILE)],
                ax_ref.at[pl.ds(coff + stripe + ROWS_PER_TILE,
                                ROWS_LAST - ROWS_PER_TILE)])

        _zero_acc()
        plsc.subcore_barrier()

    with jax.named_scope("hop2"):
        _hop(ax_ref, src4_ref, dst4_ref, acc, valbuf, sidx, didx, rows,
             gsem, ssem, xsem, dsem, c, s)
        plsc.subcore_barrier()

    a = avec[...]
    alph = 1.0 / (1.0 + jnp.exp(-a))

    def fin_chunk(rbase, nrows):
        cp_a = pltpu.async_copy(acc.at[pl.ds(rbase, nrows)],
                                rows0.at[pl.ds(0, nrows)], gsem0)
        cp_x = pltpu.async_copy(xs_ref.at[pl.ds(coff + rbase, nrows)],
                                rows1.at[pl.ds(0, nrows)], gsem1)
        cp_a.wait()
        cp_x.wait()
        for r in range(nrows):
            for q in range(DH // 16):
                sl = pl.ds(q * 16, 16)
                rows0[r, sl] = alph * rows0[r, sl] - rows1[r, sl]
        pltpu.sync_copy(rows0.at[pl.ds(0, nrows)],
                        out_ref.at[pl.ds(coff + rbase, nrows)])

    def fin48(k, carry):
        fin_chunk(pl.multiple_of(stripe + k * 48, FIN_CHUNK), 48)
        return carry

    with jax.named_scope("finalize"):
        lax.fori_loop(0, ROWS_PER_TILE // 48, fin48, 0)

        @pl.when(is_last)
        def _():
            fin_chunk(stripe + ROWS_PER_TILE, ROWS_LAST - ROWS_PER_TILE)


@jax.jit
def _diffuse(xs, src4, dst4, vals2, alpha16, zeros):
    mesh = plsc.VectorSubcoreMesh(core_axis_name="c", subcore_axis_name="s")
    f = pl.kernel(
        _body,
        mesh=mesh,
        out_type=[
            jax.ShapeDtypeStruct((NCORES * NN, DH), jnp.float32),
            jax.ShapeDtypeStruct((NCORES * NN, DH), jnp.float32),
        ],
        scratch_types=[
            pltpu.VMEM_SHARED((NN, DH), jnp.float32),
            pltpu.VMEM((EPT_PAD,), jnp.float32),
            pltpu.VMEM((1, KB), jnp.int32),
            pltpu.VMEM((1, KB), jnp.int32),
            pltpu.VMEM((1, KB), jnp.int32),
            pltpu.VMEM((1, KB), jnp.int32),
            pltpu.VMEM((1, KB), jnp.int32),
            pltpu.VMEM((1, KB), jnp.int32),
            pltpu.VMEM((KB, DH), jnp.float32),
            pltpu.VMEM((KB, DH), jnp.float32),
            pltpu.VMEM((KB, DH), jnp.float32),
            pltpu.VMEM((16,), jnp.float32),
        ] + [pltpu.SemaphoreType.DMA] * 12,
    )
    return f(xs, src4, dst4, vals2, alpha16, zeros)


def kernel(t, x, adj_indices, adj_values, alpha_train):
    del t
    n, d = x.shape
    xs = x.reshape(n, NCORES, DH).transpose(1, 0, 2).reshape(NCORES * n, DH)
    src = adj_indices[0].reshape(NSUB, EDGES_PER_TILE)
    dst = adj_indices[1].reshape(NSUB, EDGES_PER_TILE)
    vals = adj_values.reshape(NSUB, EDGES_PER_TILE)
    pad_i = jnp.zeros((NSUB, EPT_PAD - EDGES_PER_TILE), jnp.int32)
    pad_f = jnp.zeros((NSUB, EPT_PAD - EDGES_PER_TILE), jnp.float32)
    srcp = jnp.concatenate([src, pad_i], axis=1)
    dstp = jnp.concatenate([dst, pad_i], axis=1)
    vals2 = jnp.concatenate([vals, pad_f], axis=1)
    src4 = jnp.stack([srcp, srcp + n]).reshape(NCORES * NSUB * NBATCH, 1, KB)
    dst4 = dstp.reshape(NSUB * NBATCH, 1, KB)
    alpha16 = jnp.broadcast_to(alpha_train.astype(jnp.float32), (16,))
    zeros = jnp.zeros((ROWS_PER_TILE, DH), jnp.float32)
    out, _ = _diffuse(xs, src4, dst4, vals2, alpha16, zeros)
    return out.reshape(NCORES, n, DH).transpose(1, 0, 2).reshape(n, d)
